# baseline (speedup 1.0000x reference)
import functools

import jax
import jax.numpy as jnp
from jax import lax
from jax.experimental import pallas as pl
from jax.experimental.pallas import tpu as pltpu
from jax.experimental.pallas import tpu_sc as plsc

_PAD_ID = 0
_UNK_ID = 1
_OFFSET = 32000
_EPS = 1e-20

_B, _T, _V = 2, 2048, 32104
_R = _B * _T
_L = 16
_NC, _NS = 2, 16
_NW = _NC * _NS
_RPW = _R // _NW
_NCHUNK = _RPW // _L


def _sc_final_prob(table, al, tg):
    mesh = plsc.VectorSubcoreMesh(core_axis_name="c", subcore_axis_name="s")

    @functools.partial(
        pl.kernel,
        out_type=jax.ShapeDtypeStruct((_R,), jnp.float32),
        mesh=mesh,
        scratch_types=[
            pltpu.VMEM((_RPW,), jnp.int32),
            pltpu.VMEM((_RPW,), jnp.int32),
            pltpu.VMEM((_RPW,), jnp.int32),
            pltpu.VMEM((_RPW,), jnp.int32),
            pltpu.VMEM((_RPW,), jnp.float32),
            pltpu.VMEM((_RPW,), jnp.float32),
            pltpu.VMEM((_RPW,), jnp.float32),
            pltpu.SemaphoreType.DMA,
        ],
    )
    def k(table_hbm, al_hbm, tg_hbm, out_hbm,
          al_v, tg_v, idx1_v, idx2_v, val1_v, val2_v, out_v, sem):
        wid = lax.axis_index("s") * _NC + lax.axis_index("c")
        base = wid * _RPW
        pltpu.sync_copy(al_hbm.at[pl.ds(base, _RPW)], al_v)
        pltpu.sync_copy(tg_hbm.at[pl.ds(base, _RPW)], tg_v)

        for j in range(_NCHUNK):
            sl = pl.ds(j * _L, _L)
            row = (base + j * _L) + lax.iota(jnp.int32, _L)
            row_base = row * _V
            idx1_v[sl] = row_base + al_v[sl] + _OFFSET
            idx2_v[sl] = row_base + tg_v[sl]

        c1 = pltpu.async_copy(table_hbm.at[idx1_v], val1_v, sem)
        c2 = pltpu.async_copy(table_hbm.at[idx2_v], val2_v, sem)
        c1.wait()
        c2.wait()

        for j in range(_NCHUNK):
            sl = pl.ds(j * _L, _L)
            alc = al_v[sl]
            tgc = tg_v[sl]
            g1 = val1_v[sl]
            g2 = val2_v[sl]
            al_unk = alc == _UNK_ID
            tg_unk = tgc == _UNK_ID
            extra = jnp.where(al_unk, 0.0, g1) + _EPS
            fp = extra + jnp.where(tg_unk, 0.0, g2)
            fp = fp + jnp.where(al_unk & tg_unk, g2, 0.0)
            out_v[sl] = jnp.where(tgc == _PAD_ID, 1.0, fp)

        pltpu.sync_copy(out_v, out_hbm.at[pl.ds(base, _RPW)])

    return k(table, al, tg)


def _tc_neg_log_sum(fp):
    def body(fp_ref, out_ref):
        out_ref[0, 0] = -jnp.sum(jnp.log(fp_ref[...]))

    out = pl.pallas_call(
        body,
        out_shape=jax.ShapeDtypeStruct((1, 1), jnp.float32),
        in_specs=[pl.BlockSpec(memory_space=pltpu.VMEM)],
        out_specs=pl.BlockSpec(memory_space=pltpu.SMEM),
    )(fp.reshape(_R // 128, 128))
    return out[0, 0]


def kernel(prob, alignment, target):
    fp = _sc_final_prob(prob.reshape(-1), alignment.reshape(-1),
                        target.reshape(-1))
    return _tc_neg_log_sum(fp)


# speedup vs baseline: 2.7248x; 2.7248x over previous
import functools

import jax
import jax.numpy as jnp
from jax import lax
from jax.experimental import pallas as pl
from jax.experimental.pallas import tpu as pltpu
from jax.experimental.pallas import tpu_sc as plsc

_PAD_ID = 0
_UNK_ID = 1
_OFFSET = 32000
_EPS = 1e-20

_B, _T, _V = 2, 2048, 32104
_R = _B * _T
_L = 16
_NC, _NS = 2, 16
_NW = _NC * _NS
_RPW = _R // _NW
_NE = _V - _OFFSET
_HP = _RPW // 2


def _sc_final_prob(prob2d, al, tg):
    mesh = plsc.VectorSubcoreMesh(core_axis_name="c", subcore_axis_name="s")

    @functools.partial(
        pl.kernel,
        out_type=jax.ShapeDtypeStruct((_R,), jnp.float32),
        mesh=mesh,
        scratch_types=[
            pltpu.VMEM((_RPW,), jnp.int32),
            pltpu.VMEM((_RPW,), jnp.int32),
            pltpu.VMEM((_RPW, _NE), jnp.float32),
            pltpu.VMEM((_HP * 8, 128), jnp.float32),
            pltpu.VMEM((_RPW,), jnp.float32),
            pltpu.SemaphoreType.DMA,
            pltpu.SemaphoreType.DMA,
        ],
        compiler_params=pltpu.CompilerParams(use_tc_tiling_on_sc=True,
                                             needs_layout_passes=False),
    )
    def k(prob_hbm, al_hbm, tg_hbm, out_hbm,
          al_v, tg_v, ebuf_v, tbuf_v, out_v, sem_e, sem_o):
        wid = lax.axis_index("s") * _NC + lax.axis_index("c")
        base = pl.multiple_of(wid * _RPW, 8)
        pltpu.sync_copy(al_hbm.at[pl.ds(base, _RPW)], al_v)
        pltpu.sync_copy(tg_hbm.at[pl.ds(base, _RPW)], tg_v)

        ec = pltpu.async_copy(
            prob_hbm.at[pl.ds(base, _RPW), pl.ds(_OFFSET, _NE)], ebuf_v, sem_e)

        for h in range(2):
            for jc in range(_HP // _L):
                tgc = tg_v[pl.ds((h * (_HP // _L) + jc) * _L, _L)]
                for kk in range(_L):
                    i = jc * _L + kk
                    rg = pl.multiple_of(base + ((h * _HP + i) & -8), 8)
                    c2 = pl.multiple_of(tgc[kk] & -128, 128)
                    pltpu.async_copy(
                        prob_hbm.at[pl.ds(rg, 8), pl.ds(c2, 128)],
                        tbuf_v.at[pl.ds(i * 8, 8), :], sem_o)
            pltpu.make_async_copy(
                prob_hbm.at[pl.ds(0, _HP * 8), pl.ds(0, 128)], tbuf_v,
                sem_o).wait()
            if h == 0:
                ec.wait()

            for j in range(_HP // _L):
                jj = h * (_HP // _L) + j
                sl = pl.ds(jj * _L, _L)
                iota = lax.iota(jnp.int32, _L)
                i_loc = j * _L + iota
                alc = al_v[sl]
                tgc = tg_v[sl]
                g1 = plsc.load_gather(ebuf_v, [jj * _L + iota, alc])
                g2 = plsc.load_gather(
                    tbuf_v, [i_loc * 8 + (iota & 7), tgc & 127])
                al_unk = alc == _UNK_ID
                tg_unk = tgc == _UNK_ID
                extra = jnp.where(al_unk, 0.0, g1) + _EPS
                fp = extra + jnp.where(tg_unk, 0.0, g2)
                fp = fp + jnp.where(al_unk & tg_unk, g2, 0.0)
                out_v[sl] = jnp.where(tgc == _PAD_ID, 1.0, fp)

        pltpu.sync_copy(out_v, out_hbm.at[pl.ds(base, _RPW)])

    return k(prob2d, al, tg)


def _tc_neg_log_sum(fp):
    def body(fp_ref, out_ref):
        out_ref[0, 0] = -jnp.sum(jnp.log(fp_ref[...]))

    out = pl.pallas_call(
        body,
        out_shape=jax.ShapeDtypeStruct((1, 1), jnp.float32),
        in_specs=[pl.BlockSpec(memory_space=pltpu.VMEM)],
        out_specs=pl.BlockSpec(memory_space=pltpu.SMEM),
    )(fp.reshape(_R // 128, 128))
    return out[0, 0]


def kernel(prob, alignment, target):
    fp = _sc_final_prob(prob.reshape(_R, _V), alignment.reshape(-1),
                        target.reshape(-1))
    return _tc_neg_log_sum(fp)


# speedup vs baseline: 35.1235x; 12.8905x over previous
import functools

import jax
import jax.numpy as jnp
from jax import lax
from jax.experimental import pallas as pl
from jax.experimental.pallas import tpu as pltpu
from jax.experimental.pallas import tpu_sc as plsc

_PAD_ID = 0
_UNK_ID = 1
_OFFSET = 32000
_EPS = 1e-20

_B, _T, _V = 2, 2048, 32104
_R = _B * _T
_L = 16
_NC, _NS = 2, 16
_NW = _NC * _NS
_RPW = _R // _NW
_NE = _V - _OFFSET
_HP = _RPW // 2


def _sc_final_prob(probt, al, tg):
    mesh = plsc.VectorSubcoreMesh(core_axis_name="c", subcore_axis_name="s")

    @functools.partial(
        pl.kernel,
        out_type=jax.ShapeDtypeStruct((_R,), jnp.float32),
        mesh=mesh,
        scratch_types=[
            pltpu.VMEM((_RPW,), jnp.int32),
            pltpu.VMEM((_RPW,), jnp.int32),
            pltpu.VMEM((_NE, 128), jnp.float32),
            pltpu.VMEM((_HP * 8, 128), jnp.float32),
            pltpu.VMEM((_RPW,), jnp.float32),
            pltpu.SemaphoreType.DMA,
            pltpu.SemaphoreType.DMA,
        ],
        compiler_params=pltpu.CompilerParams(use_tc_tiling_on_sc=True,
                                             needs_layout_passes=False),
    )
    def k(prob_hbm, al_hbm, tg_hbm, out_hbm,
          al_v, tg_v, ebuf_v, tbuf_v, out_v, sem_e, sem_o):
        wid = lax.axis_index("s") * _NC + lax.axis_index("c")
        b = wid // (_NW // _B)
        t0 = pl.multiple_of((wid % (_NW // _B)) * _RPW, 128)
        base = pl.multiple_of(wid * _RPW, 8)
        pltpu.sync_copy(al_hbm.at[pl.ds(base, _RPW)], al_v)
        pltpu.sync_copy(tg_hbm.at[pl.ds(base, _RPW)], tg_v)

        ec = pltpu.async_copy(
            prob_hbm.at[b, pl.ds(_OFFSET, _NE), pl.ds(t0, 128)], ebuf_v, sem_e)

        for h in range(2):
            for jc in range(_HP // _L):
                tgc = tg_v[pl.ds((h * (_HP // _L) + jc) * _L, _L)]
                for kk in range(_L):
                    i = jc * _L + kk
                    v0 = pl.multiple_of(tgc[kk] & -8, 8)
                    pltpu.async_copy(
                        prob_hbm.at[b, pl.ds(v0, 8), pl.ds(t0, 128)],
                        tbuf_v.at[pl.ds(i * 8, 8), :], sem_o)

            pltpu.make_async_copy(
                prob_hbm.at[b, pl.ds(0, _HP * 8), pl.ds(0, 128)], tbuf_v,
                sem_o).wait()
            if h == 0:
                ec.wait()

            for j in range(_HP // _L):
                jj = h * (_HP // _L) + j
                sl = pl.ds(jj * _L, _L)
                iota = lax.iota(jnp.int32, _L)
                col = jj * _L + iota
                alc = al_v[sl]
                tgc = tg_v[sl]
                g1 = plsc.load_gather(ebuf_v, [alc, col])
                g2 = plsc.load_gather(
                    tbuf_v, [(j * _L + iota) * 8 + (tgc & 7), col])
                al_unk = alc == _UNK_ID
                tg_unk = tgc == _UNK_ID
                extra = jnp.where(al_unk, 0.0, g1) + _EPS
                fp = extra + jnp.where(tg_unk, 0.0, g2)
                fp = fp + jnp.where(al_unk & tg_unk, g2, 0.0)
                out_v[sl] = jnp.where(tgc == _PAD_ID, 1.0, fp)

        pltpu.sync_copy(out_v, out_hbm.at[pl.ds(base, _RPW)])

    return k(probt, al, tg)


def _tc_neg_log_sum(fp):
    def body(fp_ref, out_ref):
        out_ref[0, 0] = -jnp.sum(jnp.log(fp_ref[...]))

    out = pl.pallas_call(
        body,
        out_shape=jax.ShapeDtypeStruct((1, 1), jnp.float32),
        in_specs=[pl.BlockSpec(memory_space=pltpu.VMEM)],
        out_specs=pl.BlockSpec(memory_space=pltpu.SMEM),
    )(fp.reshape(_R // 128, 128))
    return out[0, 0]


def kernel(prob, alignment, target):
    fp = _sc_final_prob(jnp.swapaxes(prob, 1, 2), alignment.reshape(-1),
                        target.reshape(-1))
    return _tc_neg_log_sum(fp)


# speedup vs baseline: 36.9749x; 1.0527x over previous
import functools

import jax
import jax.numpy as jnp
from jax import lax
from jax.experimental import pallas as pl
from jax.experimental.pallas import tpu as pltpu
from jax.experimental.pallas import tpu_sc as plsc

_PAD_ID = 0
_UNK_ID = 1
_OFFSET = 32000
_EPS = 1e-20

_B, _T, _V = 2, 2048, 32104
_R = _B * _T
_L = 16
_NC, _NS = 2, 16
_NW = _NC * _NS
_RPW = _R // _NW
_NE = _V - _OFFSET
_HP = _RPW // 2
_VQ = _V // 8
_TQ = _T // 128
_EQ = _OFFSET // 8


def _sc_final_prob(probq, al, tg):
    mesh = plsc.VectorSubcoreMesh(core_axis_name="c", subcore_axis_name="s")

    @functools.partial(
        pl.kernel,
        out_type=jax.ShapeDtypeStruct((_R,), jnp.float32),
        mesh=mesh,
        scratch_types=[
            pltpu.VMEM((_RPW,), jnp.int32),
            pltpu.VMEM((_RPW,), jnp.int32),
            pltpu.VMEM((_L,), jnp.int32),
            pltpu.VMEM((_HP,), jnp.int32),
            pltpu.VMEM((_L, 8, 128), jnp.float32),
            pltpu.VMEM((_HP, 8, 128), jnp.float32),
            pltpu.VMEM((_RPW,), jnp.float32),
            pltpu.SemaphoreType.DMA,
            pltpu.SemaphoreType.DMA,
        ],
        compiler_params=pltpu.CompilerParams(use_tc_tiling_on_sc=True,
                                             needs_layout_passes=False),
    )
    def k(probq_hbm, al_hbm, tg_hbm, out_hbm,
          al_v, tg_v, eidx_v, idx_v, ebuf_v, tbuf_v, out_v, sem_e, sem_o):
        wid = lax.axis_index("s") * _NC + lax.axis_index("c")
        b = wid // (_NW // _B)
        tq = wid % (_NW // _B)
        base = pl.multiple_of(wid * _RPW, 8)
        pltpu.sync_copy(al_hbm.at[pl.ds(base, _RPW)], al_v)
        pltpu.sync_copy(tg_hbm.at[pl.ds(base, _RPW)], tg_v)

        iota = lax.iota(jnp.int32, _L)
        eidx_v[...] = (b * _VQ + _EQ + jnp.minimum(iota, _NE // 8 - 1)) * _TQ + tq
        ec = pltpu.async_copy(probq_hbm.at[eidx_v], ebuf_v, sem_e)

        for h in range(2):
            for jc in range(_HP // _L):
                sl16 = pl.ds((h * (_HP // _L) + jc) * _L, _L)
                tgc = tg_v[sl16]
                idx_v[pl.ds(jc * _L, _L)] = (b * _VQ + (tgc >> 3)) * _TQ + tq

            oc = pltpu.async_copy(probq_hbm.at[idx_v], tbuf_v, sem_o)
            if h == 0:
                ec.wait()
            oc.wait()

            for j in range(_HP // _L):
                jj = h * (_HP // _L) + j
                sl = pl.ds(jj * _L, _L)
                col = jj * _L + iota
                alc = al_v[sl]
                tgc = tg_v[sl]
                g1 = plsc.load_gather(ebuf_v, [alc >> 3, alc & 7, col])
                g2 = plsc.load_gather(tbuf_v, [j * _L + iota, tgc & 7, col])
                al_unk = alc == _UNK_ID
                tg_unk = tgc == _UNK_ID
                extra = jnp.where(al_unk, 0.0, g1) + _EPS
                fp = extra + jnp.where(tg_unk, 0.0, g2)
                fp = fp + jnp.where(al_unk & tg_unk, g2, 0.0)
                out_v[sl] = jnp.where(tgc == _PAD_ID, 1.0, fp)

        pltpu.sync_copy(out_v, out_hbm.at[pl.ds(base, _RPW)])

    return k(probq, al, tg)


def _tc_neg_log_sum(fp):
    def body(fp_ref, out_ref):
        out_ref[0, 0] = -jnp.sum(jnp.log(fp_ref[...]))

    out = pl.pallas_call(
        body,
        out_shape=jax.ShapeDtypeStruct((1, 1), jnp.float32),
        in_specs=[pl.BlockSpec(memory_space=pltpu.VMEM)],
        out_specs=pl.BlockSpec(memory_space=pltpu.SMEM),
    )(fp.reshape(_R // 128, 128))
    return out[0, 0]


def kernel(prob, alignment, target):
    probq = prob.reshape(_B, _TQ, 128, _VQ, 8).transpose(0, 3, 1, 4, 2)
    probq = probq.reshape(_B * _VQ * _TQ, 8, 128)
    fp = _sc_final_prob(probq, alignment.reshape(-1), target.reshape(-1))
    return _tc_neg_log_sum(fp)


# speedup vs baseline: 48.6337x; 1.3153x over previous
import functools

import jax
import jax.numpy as jnp
from jax import lax
from jax.experimental import pallas as pl
from jax.experimental.pallas import tpu as pltpu
from jax.experimental.pallas import tpu_sc as plsc

_PAD_ID = 0
_UNK_ID = 1
_OFFSET = 32000
_EPS = 1e-20

_B, _T, _V = 2, 2048, 32104
_R = _B * _T
_L = 16
_NC, _NS = 2, 16
_NW = _NC * _NS
_RPW = _R // _NW
_VQ = _V // 8
_TQ = _T // 128


def _sc_final_prob(probf, al, tg):
    mesh = plsc.VectorSubcoreMesh(core_axis_name="c", subcore_axis_name="s")

    @functools.partial(
        pl.kernel,
        out_type=jax.ShapeDtypeStruct((_R,), jnp.float32),
        mesh=mesh,
        scratch_types=[
            pltpu.VMEM((_RPW,), jnp.int32),
            pltpu.VMEM((_RPW,), jnp.int32),
            pltpu.VMEM((_RPW,), jnp.int32),
            pltpu.VMEM((_RPW,), jnp.int32),
            pltpu.VMEM((_RPW,), jnp.float32),
            pltpu.VMEM((_RPW,), jnp.float32),
            pltpu.VMEM((_RPW,), jnp.float32),
            pltpu.SemaphoreType.DMA,
        ],
    )
    def k(probf_hbm, al_hbm, tg_hbm, out_hbm,
          al_v, tg_v, idx1_v, idx2_v, val1_v, val2_v, out_v, sem):
        wid = lax.axis_index("s") * _NC + lax.axis_index("c")
        b = wid // (_NW // _B)
        tq = wid % (_NW // _B)
        base = wid * _RPW
        pltpu.sync_copy(al_hbm.at[pl.ds(base, _RPW)], al_v)
        pltpu.sync_copy(tg_hbm.at[pl.ds(base, _RPW)], tg_v)

        tile0 = (b * _VQ * _TQ + tq) * 1024
        for j in range(_RPW // _L):
            sl = pl.ds(j * _L, _L)
            col = j * _L + lax.iota(jnp.int32, _L)
            ve = al_v[sl] + _OFFSET
            vo = tg_v[sl]
            idx1_v[sl] = tile0 + (ve >> 3) * (_TQ * 1024) + (ve & 7) * 128 + col
            idx2_v[sl] = tile0 + (vo >> 3) * (_TQ * 1024) + (vo & 7) * 128 + col

        c1 = pltpu.async_copy(probf_hbm.at[idx1_v], val1_v, sem)
        c2 = pltpu.async_copy(probf_hbm.at[idx2_v], val2_v, sem)
        c1.wait()
        c2.wait()

        for j in range(_RPW // _L):
            sl = pl.ds(j * _L, _L)
            alc = al_v[sl]
            tgc = tg_v[sl]
            g1 = val1_v[sl]
            g2 = val2_v[sl]
            al_unk = alc == _UNK_ID
            tg_unk = tgc == _UNK_ID
            extra = jnp.where(al_unk, 0.0, g1) + _EPS
            fp = extra + jnp.where(tg_unk, 0.0, g2)
            fp = fp + jnp.where(al_unk & tg_unk, g2, 0.0)
            out_v[sl] = jnp.where(tgc == _PAD_ID, 1.0, fp)

        pltpu.sync_copy(out_v, out_hbm.at[pl.ds(base, _RPW)])

    return k(probf, al, tg)


def _tc_neg_log_sum(fp):
    def body(fp_ref, out_ref):
        out_ref[0, 0] = -jnp.sum(jnp.log(fp_ref[...]))

    out = pl.pallas_call(
        body,
        out_shape=jax.ShapeDtypeStruct((1, 1), jnp.float32),
        in_specs=[pl.BlockSpec(memory_space=pltpu.VMEM)],
        out_specs=pl.BlockSpec(memory_space=pltpu.SMEM),
    )(fp.reshape(_R // 128, 128))
    return out[0, 0]


def kernel(prob, alignment, target):
    probf = prob.reshape(_B, _TQ, 128, _VQ, 8).transpose(0, 3, 1, 4, 2)
    probf = probf.reshape(_B * _V * _T)
    fp = _sc_final_prob(probf, alignment.reshape(-1), target.reshape(-1))
    return _tc_neg_log_sum(fp)
